# Initial kernel scaffold; baseline (speedup 1.0000x reference)
import jax
import jax.numpy as jnp
from jax.experimental import pallas as pl


def kernel(x, old_table, new_table):
    raise NotImplementedError("write your pallas kernel here")


# speedup vs baseline: 2.4535x; 2.4535x over previous
import functools

import jax
import jax.numpy as jnp
from jax import lax
from jax.experimental import pallas as pl
from jax.experimental.pallas import tpu as pltpu
from jax.experimental.pallas import tpu_sc as plsc

OLD_V = 1000000
NEW_V = 1024
D = 64
N = 4096 * 200
NC = 2
NS = 16
NW = NC * NS
PER_W = N // NW
CH = 128
NCH = PER_W // CH
GP = CH // 16


def _body(x_hbm, old_hbm, new_hbm, out_hbm, x_all, idx_c, rows_v, newt_v, sem):
    wid = lax.axis_index("s") * NC + lax.axis_index("c")
    base = pl.multiple_of(wid * PER_W, PER_W)
    pltpu.sync_copy(new_hbm, newt_v)
    pltpu.sync_copy(x_hbm.at[pl.ds(base, PER_W)], x_all)
    lane = lax.iota(jnp.int32, 16)

    def chunk(c, carry0):
        cb = pl.multiple_of(c * CH, CH)
        for g in range(GP):
            xv = x_all[pl.ds(cb + g * 16, 16)]
            idx_c[pl.ds(g * 16, 16)] = jnp.minimum(xv, OLD_V - 1)
        pltpu.async_copy(old_hbm.at[idx_c], rows_v, sem).wait()
        for g in range(GP):
            xv = x_all[pl.ds(cb + g * 16, 16)]
            m = xv >= OLD_V
            cnt = plsc.all_reduce_population_count(m)

            @pl.when(cnt[0] > 0)
            def _():
                nid = jnp.maximum(xv - OLD_V, 0)
                tok = lane + g * 16
                for d in range(D):
                    col = jnp.full((16,), d, jnp.int32)
                    vals = plsc.load_gather(newt_v, [nid, col])
                    plsc.store_scatter(rows_v, [tok, col], vals, mask=m)
        pltpu.sync_copy(rows_v, out_hbm.at[pl.ds(base + cb, CH)])
        return carry0

    lax.fori_loop(0, NCH, chunk, 0)


def kernel(x, old_table, new_table):
    xf = x.reshape(-1)
    mesh = plsc.VectorSubcoreMesh(core_axis_name="c", subcore_axis_name="s")
    run = functools.partial(
        pl.kernel,
        mesh=mesh,
        out_type=jax.ShapeDtypeStruct((N, D), jnp.float32),
        scratch_types=[
            pltpu.VMEM((PER_W,), jnp.int32),
            pltpu.VMEM((CH,), jnp.int32),
            pltpu.VMEM((CH, D), jnp.float32),
            pltpu.VMEM((NEW_V, D), jnp.float32),
            pltpu.SemaphoreType.DMA,
        ],
        compiler_params=pltpu.CompilerParams(
            needs_layout_passes=False, use_tc_tiling_on_sc=False),
    )(_body)
    out = run(xf, old_table, new_table)
    return out.reshape(x.shape[0], x.shape[1], D)


# speedup vs baseline: 2.8268x; 1.1521x over previous
import functools

import jax
import jax.numpy as jnp
from jax import lax
from jax.experimental import pallas as pl
from jax.experimental.pallas import tpu as pltpu
from jax.experimental.pallas import tpu_sc as plsc

OLD_V = 1000000
NEW_V = 1024
D = 64
N = 4096 * 200
NC = 2
NS = 16
NW = NC * NS
PER_W = N // NW
CH = 128
NCH = PER_W // CH
GP = CH // 16
NBUF = 4
LEAD = 2


def _body(x_hbm, old_hbm, new_hbm, out_hbm, x_all, idxb, rows, newt_v,
          gsem, osem):
    wid = lax.axis_index("s") * NC + lax.axis_index("c")
    base = pl.multiple_of(wid * PER_W, PER_W)
    pltpu.sync_copy(new_hbm, newt_v)
    pltpu.sync_copy(x_hbm.at[pl.ds(base, PER_W)], x_all)
    lane = lax.iota(jnp.int32, 16)

    def gather_desc(c, b):
        idx_slice = idxb.at[pl.ds(pl.multiple_of(b * CH, CH), CH)]
        return pltpu.make_async_copy(old_hbm.at[idx_slice], rows.at[b],
                                     gsem.at[b])

    def store_desc(c, b):
        return pltpu.make_async_copy(
            rows.at[b], out_hbm.at[pl.ds(base + c * CH, CH)], osem.at[b])

    def fire(c):
        b = lax.rem(c, NBUF)
        cb = pl.multiple_of(c * CH, CH)
        bb = pl.multiple_of(b * CH, CH)
        for g in range(GP):
            xv = x_all[pl.ds(cb + g * 16, 16)]
            idxb[pl.ds(bb + g * 16, 16)] = jnp.minimum(xv, OLD_V - 1)
        gather_desc(c, b).start()

    for c0 in range(LEAD):
        fire(jnp.int32(c0))

    def step(c, carry0):
        c2 = c + LEAD
        b2 = lax.rem(c2, NBUF)

        @pl.when(c2 < NCH)
        def _():
            @pl.when(c2 >= NBUF)
            def _():
                store_desc(c2 - NBUF, b2).wait()
            fire(c2)

        b = lax.rem(c, NBUF)
        cb = pl.multiple_of(c * CH, CH)
        gather_desc(c, b).wait()
        bsp = jnp.full((16,), b, jnp.int32)
        for g in range(GP):
            xv = x_all[pl.ds(cb + g * 16, 16)]
            m = xv >= OLD_V
            cnt = plsc.all_reduce_population_count(m)

            @pl.when(cnt[0] > 0)
            def _():
                nid = jnp.maximum(xv - OLD_V, 0)
                tok = lane + g * 16
                for d in range(D):
                    col = jnp.full((16,), d, jnp.int32)
                    vals = plsc.load_gather(newt_v, [nid, col])
                    plsc.store_scatter(rows, [bsp, tok, col], vals, mask=m)
        store_desc(c, b).start()
        return carry0

    lax.fori_loop(0, NCH, step, 0)

    for k in range(NBUF):
        c = jnp.int32(NCH - NBUF + k)
        store_desc(c, lax.rem(c, NBUF)).wait()


def kernel(x, old_table, new_table):
    xf = x.reshape(-1)
    mesh = plsc.VectorSubcoreMesh(core_axis_name="c", subcore_axis_name="s")
    run = functools.partial(
        pl.kernel,
        mesh=mesh,
        out_type=jax.ShapeDtypeStruct((N, D), jnp.float32),
        scratch_types=[
            pltpu.VMEM((PER_W,), jnp.int32),
            pltpu.VMEM((NBUF * CH,), jnp.int32),
            pltpu.VMEM((NBUF, CH, D), jnp.float32),
            pltpu.VMEM((NEW_V, D), jnp.float32),
            pltpu.SemaphoreType.DMA((NBUF,)),
            pltpu.SemaphoreType.DMA((NBUF,)),
        ],
        compiler_params=pltpu.CompilerParams(
            needs_layout_passes=False, use_tc_tiling_on_sc=False),
    )(_body)
    out = run(xf, old_table, new_table)
    return out.reshape(x.shape[0], x.shape[1], D)
